# baseline (speedup 1.0000x reference)
import functools

import jax
import jax.numpy as jnp
from jax.experimental import pallas as pl
from jax.experimental.pallas import tpu as pltpu

_TM = 1024


def _vq_body(h_ref, wp_ref, bp_ref, emb_ref, wi_ref, bi_ref,
             out_ref, code_ref, embn_ref, embn16_ref, ones16_ref, dma_sem):
    @pl.when(pl.program_id(0) == 0)
    def _():
        cp = pltpu.make_async_copy(emb_ref, embn_ref, dma_sem)
        cp.start()
        cp.wait()
        e = embn_ref[...]
        en = e / jnp.sqrt(jnp.sum(e * e, axis=1, keepdims=True))
        embn_ref[...] = en
        embn16_ref[...] = en.astype(jnp.bfloat16)
        ones16_ref[...] = jnp.ones(ones16_ref.shape, jnp.bfloat16)

    h1 = jax.lax.dot_general(
        h_ref[...], wp_ref[...],
        dimension_numbers=(((1,), (1,)), ((), ())),
        preferred_element_type=jnp.float32,
    ) + bp_ref[...]
    h1 = h1 * (2.0 / jnp.sqrt(jnp.sum(h1 * h1, axis=1, keepdims=True)))

    logits2 = jax.lax.dot_general(
        h1, embn_ref[...],
        dimension_numbers=(((1,), (1,)), ((), ())),
        preferred_element_type=jnp.float32,
    )

    code_ref[...] = jnp.argmax(logits2, axis=1).astype(jnp.int32)

    e16 = jnp.exp(logits2).astype(jnp.bfloat16)
    s = jax.lax.dot_general(
        e16, ones16_ref[...],
        dimension_numbers=(((1,), (0,)), ((), ())),
        preferred_element_type=jnp.float32,
    )[:, 0:1]

    hv = jax.lax.dot_general(
        e16, embn16_ref[...],
        dimension_numbers=(((1,), (0,)), ((), ())),
        preferred_element_type=jnp.float32,
    ) / s

    out_ref[...] = jax.lax.dot_general(
        hv, wi_ref[...],
        dimension_numbers=(((1,), (1,)), ((), ())),
        preferred_element_type=jnp.float32,
    ) + bi_ref[...]


@functools.partial(jax.jit, static_argnames=())
def kernel(h, attn_mask, W_proj, b_proj, emb, W_inv, b_inv):
    B, S, Dh = h.shape
    N = B * S
    D = W_proj.shape[0]
    K = emb.shape[0]
    h2 = h.reshape(N, Dh)

    grid = (N // _TM,)
    out, code = pl.pallas_call(
        _vq_body,
        grid=grid,
        in_specs=[
            pl.BlockSpec((_TM, Dh), lambda i: (i, 0)),
            pl.BlockSpec((D, Dh), lambda i: (0, 0)),
            pl.BlockSpec((1, D), lambda i: (0, 0)),
            pl.BlockSpec(memory_space=pl.ANY),
            pl.BlockSpec((Dh, D), lambda i: (0, 0)),
            pl.BlockSpec((1, Dh), lambda i: (0, 0)),
        ],
        out_specs=[
            pl.BlockSpec((_TM, Dh), lambda i: (i, 0)),
            pl.BlockSpec((_TM,), lambda i: (i,)),
        ],
        out_shape=[
            jax.ShapeDtypeStruct((N, Dh), jnp.float32),
            jax.ShapeDtypeStruct((N,), jnp.int32),
        ],
        scratch_shapes=[pltpu.VMEM((K, D), jnp.float32),
                        pltpu.VMEM((K, D), jnp.bfloat16),
                        pltpu.VMEM((K, 128), jnp.bfloat16),
                        pltpu.SemaphoreType.DMA],
    )(h2, W_proj, b_proj.reshape(1, D), emb, W_inv, b_inv.reshape(1, Dh))

    quantized = out.reshape(B, S, Dh)
    vq_code = code.reshape(B, S).astype(jnp.int64)
    vq_loss = jnp.float32(0.0)
    return (quantized, vq_code, vq_loss)


# speedup vs baseline: 1.3520x; 1.3520x over previous
import functools

import jax
import jax.numpy as jnp
from jax.experimental import pallas as pl
from jax.experimental.pallas import tpu as pltpu

_TM = 1024


def _vq_body(h_ref, wp_ref, bp_ref, emb_ref, wi_ref, bi_ref,
             out_ref, code_ref, embn_ref, embn16_ref):
    @pl.when(pl.program_id(0) == 0)
    def _():
        e = emb_ref[...]
        en = e / jnp.sqrt(jnp.sum(e * e, axis=1, keepdims=True))
        embn_ref[...] = en
        embn16_ref[...] = en.astype(jnp.bfloat16)

    h1 = jax.lax.dot_general(
        h_ref[...], wp_ref[...],
        dimension_numbers=(((1,), (1,)), ((), ())),
        preferred_element_type=jnp.float32,
    ) + bp_ref[...]
    h1 = h1 * (2.0 / jnp.sqrt(jnp.sum(h1 * h1, axis=1, keepdims=True)))

    logits2 = jax.lax.dot_general(
        h1, embn_ref[...],
        dimension_numbers=(((1,), (1,)), ((), ())),
        preferred_element_type=jnp.float32,
    )

    code_ref[...] = jnp.argmax(logits2, axis=1).astype(jnp.int32)

    e = jnp.exp(logits2)
    s = jnp.sum(e, axis=1, keepdims=True)

    hv = jax.lax.dot_general(
        e.astype(jnp.bfloat16), embn16_ref[...],
        dimension_numbers=(((1,), (0,)), ((), ())),
        preferred_element_type=jnp.float32,
    ) / s

    out_ref[...] = jax.lax.dot_general(
        hv, wi_ref[...],
        dimension_numbers=(((1,), (1,)), ((), ())),
        preferred_element_type=jnp.float32,
    ) + bi_ref[...]


@functools.partial(jax.jit, static_argnames=())
def kernel(h, attn_mask, W_proj, b_proj, emb, W_inv, b_inv):
    B, S, Dh = h.shape
    N = B * S
    D = W_proj.shape[0]
    K = emb.shape[0]
    h2 = h.reshape(N, Dh)

    grid = (N // _TM,)
    out, code = pl.pallas_call(
        _vq_body,
        grid=grid,
        in_specs=[
            pl.BlockSpec((_TM, Dh), lambda i: (i, 0)),
            pl.BlockSpec((D, Dh), lambda i: (0, 0)),
            pl.BlockSpec((1, D), lambda i: (0, 0)),
            pl.BlockSpec((K, D), lambda i: (0, 0)),
            pl.BlockSpec((Dh, D), lambda i: (0, 0)),
            pl.BlockSpec((1, Dh), lambda i: (0, 0)),
        ],
        out_specs=[
            pl.BlockSpec((_TM, Dh), lambda i: (i, 0)),
            pl.BlockSpec((_TM,), lambda i: (i,)),
        ],
        out_shape=[
            jax.ShapeDtypeStruct((N, Dh), jnp.float32),
            jax.ShapeDtypeStruct((N,), jnp.int32),
        ],
        scratch_shapes=[pltpu.VMEM((K, D), jnp.float32),
                        pltpu.VMEM((K, D), jnp.bfloat16)],
    )(h2, W_proj, b_proj.reshape(1, D), emb, W_inv, b_inv.reshape(1, Dh))

    quantized = out.reshape(B, S, Dh)
    vq_code = code.reshape(B, S).astype(jnp.int64)
    vq_loss = jnp.float32(0.0)
    return (quantized, vq_code, vq_loss)
